# baseline (speedup 1.0000x reference)
import functools

import jax
import jax.numpy as jnp
from jax import lax
from jax.experimental import pallas as pl
from jax.experimental.pallas import tpu as pltpu
from jax.experimental.pallas import tpu_sc as plsc

_IDX_LANES = 128


def _make_gather(n_idx_rows: int, width: int):
    info = plsc.get_sparse_core_info()
    num_workers = info.num_cores * info.num_subcores
    assert n_idx_rows % num_workers == 0, (n_idx_rows, num_workers)
    rows_per_worker = n_idx_rows // num_workers
    num_cores = info.num_cores

    mesh = plsc.VectorSubcoreMesh(core_axis_name="c", subcore_axis_name="s")

    @functools.partial(
        pl.kernel,
        mesh=mesh,
        out_type=jax.ShapeDtypeStruct((n_idx_rows, _IDX_LANES, width), jnp.float32),
        scratch_types=[
            pltpu.VMEM((n_idx_rows, _IDX_LANES), jnp.int32),
            pltpu.VMEM((rows_per_worker, _IDX_LANES, width), jnp.float32),
            [pltpu.SemaphoreType.DMA] * rows_per_worker,
            pltpu.SemaphoreType.DMA,
        ],
    )
    def gather(tab_hbm, idx_hbm, out_hbm, idx_v, buf_v, gsems, ssem):
        wid = lax.axis_index("s") * num_cores + lax.axis_index("c")
        base = wid * rows_per_worker
        pltpu.sync_copy(idx_hbm, idx_v)
        gathers = [
            pltpu.async_copy(tab_hbm.at[idx_v.at[base + j]], buf_v.at[j], gsems[j])
            for j in range(rows_per_worker)
        ]
        stores = []
        for j, g in enumerate(gathers):
            g.wait()
            stores.append(pltpu.async_copy(buf_v.at[j], out_hbm.at[base + j], ssem))
        for st in stores:
            st.wait()

    return gather


def kernel(x, position_ids, cached_cos, cached_sin):
    del x
    b, s = position_ids.shape
    half = cached_cos.shape[3]
    n = b * s
    assert n % _IDX_LANES == 0, (b, s)
    table = jnp.concatenate([cached_cos[0, 0], cached_sin[0, 0]], axis=-1)
    idx = position_ids.reshape(n // _IDX_LANES, _IDX_LANES).astype(jnp.int32)
    out = _make_gather(n // _IDX_LANES, 2 * half)(table, idx)
    out = out.reshape(b, s, 1, 2 * half)
    return (out[..., :half], out[..., half:])
